# baseline (speedup 1.0000x reference)
import functools

import jax
import jax.numpy as jnp
from jax.experimental import pallas as pl
from jax.experimental.pallas import tpu as pltpu

N_NODES = 10000
N_EDGES = 160000
N_MASKED = 112000
D = 128
DE = 16
LAT = 64

NB = 1000
EB = 2048
E_P = 163840
MB = 2000


def _ln(x, g, b, eps=1e-5):
    mu = x.mean(-1, keepdims=True)
    var = ((x - mu) ** 2).mean(-1, keepdims=True)
    return (x - mu) / jnp.sqrt(var + eps) * g + b


def _silu(x):
    return x * jax.nn.sigmoid(x)


def _dot(a, b):
    return jnp.dot(a, b, preferred_element_type=jnp.float32)


def _embed_body(x, w1, b1, g1, c1, w2, b2, g2, c2, wa, wb,
                h_ref, ha_ref, hb_ref):
    h = _dot(x[...], w1[...]) + b1[...]
    h = jax.nn.gelu(_ln(h, g1[...], c1[...]))
    h = _dot(h, w2[...]) + b2[...]
    h = jax.nn.gelu(_ln(h, g2[...], c2[...]))
    h_ref[...] = h
    ha_ref[...] = _dot(h, wa[...])
    hb_ref[...] = _dot(h, wb[...])


def _full(shape):
    return pl.BlockSpec(shape, lambda i: (0,) * len(shape))


def _rows(nb, d):
    return pl.BlockSpec((nb, d), lambda i: (i, 0))


def _embed_call(x, w1, b1, g1, c1, w2, b2, g2, c2, wa, wb):
    f = pl.pallas_call(
        _embed_body,
        grid=(N_NODES // NB,),
        in_specs=[_rows(NB, D)] + [_full(a.shape) for a in
                                   (w1, b1, g1, c1, w2, b2, g2, c2, wa, wb)],
        out_specs=[_rows(NB, D)] * 3,
        out_shape=[jax.ShapeDtypeStruct((N_NODES, D), jnp.float32)] * 3,
    )
    return f(x, w1, b1, g1, c1, w2, b2, g2, c2, wa, wb)


def _make_edge_body(with_coord):
    if with_coord:
        def body(has, hbd, xs, xd, ef, wd, wc, be1, w2, b2, wc1, bc1, wc2,
                 m2_ref, mx_ref):
            row0 = pl.program_id(0) * EB
            rid = row0 + jax.lax.broadcasted_iota(jnp.int32, (EB, 1), 0)
            valid = (rid < N_EDGES).astype(jnp.float32)
            dx = xs[...] - xd[...]
            radial = jnp.sum(dx * dx, axis=-1, keepdims=True)
            m1 = _silu(has[...] + hbd[...] + radial * wc[...]
                       + _dot(ef[...], wd[...]) + be1[...])
            m2 = _silu(_dot(m1, w2[...]) + b2[...])
            t = _silu(_dot(m2, wc1[...]) + bc1[...])
            cw = jnp.sum(t * wc2[...], axis=-1, keepdims=True)
            col = jax.lax.broadcasted_iota(jnp.int32, (1, DE), 1)
            e3 = jnp.where(col == 3, 1.0, 0.0)
            m2_ref[...] = m2 * valid
            mx_ref[...] = (cw * dx + e3) * valid
        return body
    else:
        def body(has, hbd, xs, xd, ef, wd, wc, be1, w2, b2,
                 m2_ref):
            row0 = pl.program_id(0) * EB
            rid = row0 + jax.lax.broadcasted_iota(jnp.int32, (EB, 1), 0)
            valid = (rid < N_EDGES).astype(jnp.float32)
            dx = xs[...] - xd[...]
            radial = jnp.sum(dx * dx, axis=-1, keepdims=True)
            m1 = _silu(has[...] + hbd[...] + radial * wc[...]
                       + _dot(ef[...], wd[...]) + be1[...])
            m2 = _silu(_dot(m1, w2[...]) + b2[...])
            m2_ref[...] = m2 * valid
        return body


def _edge_call(with_coord, has, hbd, xs, xd, ef, wd, wc, be1, w2, b2,
               wc1=None, bc1=None, wc2=None):
    args = [has, hbd, xs, xd, ef, wd, wc, be1, w2, b2]
    n_out = 1
    if with_coord:
        args += [wc1, bc1, wc2]
        n_out = 2
    in_specs = ([_rows(EB, D), _rows(EB, D), _rows(EB, DE), _rows(EB, DE),
                 _rows(EB, DE)] + [_full(a.shape) for a in args[5:]])
    out_specs = [_rows(EB, D)] + ([_rows(EB, DE)] if with_coord else [])
    out_shape = [jax.ShapeDtypeStruct((E_P, D), jnp.float32)] + (
        [jax.ShapeDtypeStruct((E_P, DE), jnp.float32)] if with_coord else [])
    f = pl.pallas_call(
        _make_edge_body(with_coord),
        grid=(E_P // EB,),
        in_specs=in_specs,
        out_specs=out_specs if n_out > 1 else out_specs[0],
        out_shape=out_shape if n_out > 1 else out_shape[0],
    )
    return f(*args)


def _make_node_body(n_partial, final):
    if not final:
        def body(h, hs, ms, xp, wnh, wnn, bn1, wn2, bn2, g, c, wa, wb,
                 h_ref, x_ref, ha_ref, hb_ref):
            hsum = hs[0]
            msum = ms[0]
            for k in range(1, n_partial):
                hsum = hsum + hs[k]
                msum = msum + ms[k]
            cnt = jnp.maximum(msum[:, 3:4], 1.0)
            hn = _silu(_dot(h[...], wnh[...]) + _dot(hsum, wnn[...]) + bn1[...])
            hh = _dot(hn, wn2[...]) + bn2[...]
            h1 = _ln(hh, g[...], c[...])
            col = jax.lax.broadcasted_iota(jnp.int32, (1, DE), 1)
            m3 = jnp.where(col < 3, 1.0, 0.0)
            h_ref[...] = h1
            x_ref[...] = xp[...] + (msum * m3) / cnt
            ha_ref[...] = _dot(h1, wa[...])
            hb_ref[...] = _dot(h1, wb[...])
        return body
    else:
        def body(h, hs, wnh, wnn, bn1, wn2, bn2, wmu, bmu, wlv, blv, eps,
                 z_ref, mu_ref, lv_ref):
            hsum = hs[0]
            for k in range(1, n_partial):
                hsum = hsum + hs[k]
            hn = _silu(_dot(h[...], wnh[...]) + _dot(hsum, wnn[...]) + bn1[...])
            h2 = _dot(hn, wn2[...]) + bn2[...]
            mu = _dot(h2, wmu[...]) + bmu[...]
            lv = _dot(h2, wlv[...]) + blv[...]
            z_ref[...] = eps[...] * jnp.exp(lv) + mu
            mu_ref[...] = mu
            lv_ref[...] = lv
        return body


def _part(nb, d, n_partial):
    return pl.BlockSpec((n_partial, nb, d), lambda i: (0, i, 0))


def _node_call(h, hs, ms, xp, wnh, wnn, bn1, wn2, bn2, g, c, wa, wb):
    np_ = hs.shape[0]
    f = pl.pallas_call(
        _make_node_body(np_, final=False),
        grid=(N_NODES // NB,),
        in_specs=[_rows(NB, D), _part(NB, D, np_), _part(NB, DE, np_),
                  _rows(NB, DE)] + [_full(a.shape) for a in
                                    (wnh, wnn, bn1, wn2, bn2, g, c, wa, wb)],
        out_specs=[_rows(NB, D), _rows(NB, DE), _rows(NB, D), _rows(NB, D)],
        out_shape=[jax.ShapeDtypeStruct((N_NODES, D), jnp.float32),
                   jax.ShapeDtypeStruct((N_NODES, DE), jnp.float32),
                   jax.ShapeDtypeStruct((N_NODES, D), jnp.float32),
                   jax.ShapeDtypeStruct((N_NODES, D), jnp.float32)],
    )
    return f(h, hs, ms, xp, wnh, wnn, bn1, wn2, bn2, g, c, wa, wb)


def _node_final_call(h, hs, wnh, wnn, bn1, wn2, bn2, wmu, bmu, wlv, blv, eps):
    np_ = hs.shape[0]
    f = pl.pallas_call(
        _make_node_body(np_, final=True),
        grid=(N_NODES // NB,),
        in_specs=[_rows(NB, D), _part(NB, D, np_)]
        + [_full(a.shape) for a in (wnh, wnn, bn1, wn2, bn2, wmu, bmu, wlv, blv)]
        + [_rows(NB, LAT)],
        out_specs=[_rows(NB, LAT)] * 3,
        out_shape=[jax.ShapeDtypeStruct((N_NODES, LAT), jnp.float32)] * 3,
    )
    return f(h, hs, wnh, wnn, bn1, wn2, bn2, wmu, bmu, wlv, blv, eps)


def _dec_body(zsp, zdp, zsn, zdn, w0, b0, w1, b1, pos_ref, neg_ref):
    xp = zsp[...] * zdp[...]
    xn = zsn[...] * zdn[...]
    ap = jax.nn.relu(_dot(xp, w0[...]) + b0[...])
    an = jax.nn.relu(_dot(xn, w0[...]) + b0[...])
    pos_ref[...] = jax.nn.sigmoid(_dot(ap, w1[...]) + b1[...])
    neg_ref[...] = jax.nn.sigmoid(_dot(an, w1[...]) + b1[...])


def _dec_call(zsp, zdp, zsn, zdn, w0, b0, w1, b1):
    f = pl.pallas_call(
        _dec_body,
        grid=(N_MASKED // MB,),
        in_specs=[_rows(MB, LAT)] * 4 + [_full(a.shape) for a in (w0, b0, w1, b1)],
        out_specs=[_rows(MB, D)] * 2,
        out_shape=[jax.ShapeDtypeStruct((N_MASKED, D), jnp.float32)] * 2,
    )
    return f(zsp, zdp, zsn, zdn, w0, b0, w1, b1)


def kernel(node_feat, xyz, edge_feat, edge_index, masked_edges, neg_edges,
           params):
    p = params
    t = lambda w: w.T
    row = lambda v: v.reshape(1, -1)

    src = edge_index[0].astype(jnp.int32)
    dst = edge_index[1].astype(jnp.int32)
    pad = E_P - N_EDGES
    srcp = jnp.concatenate([src, jnp.zeros((pad,), jnp.int32)])
    dstp = jnp.concatenate([dst, jnp.zeros((pad,), jnp.int32)])
    efp = jnp.pad(edge_feat, ((0, pad), (0, 0)))
    xp = jnp.pad(xyz, ((0, 0), (0, DE - 3)))

    e = p['emb1']; e2 = p['emb2']
    l1, l2 = p['egnn']

    def split_e1(lp):
        W = lp['e1']['W']
        return (t(W[:, :D]), t(W[:, D:2 * D]), row(W[:, 2 * D]),
                t(W[:, 2 * D + 1:]), row(lp['e1']['b']))

    h, ha, hb = _embed_call(
        node_feat, t(e['W']), row(e['b']),
        row(p['emb_ln1']['g']), row(p['emb_ln1']['b']),
        t(e2['W']), row(e2['b']),
        row(p['emb_ln2']['g']), row(p['emb_ln2']['b']),
        t(l1['e1']['W'][:, :D]), t(l1['e1']['W'][:, D:2 * D]))

    for li, lp in enumerate((l1, l2)):
        wa_, wb_, wc_, wd_, be1_ = split_e1(lp)
        has = jnp.take(ha, srcp, axis=0)
        hbd = jnp.take(hb, dstp, axis=0)
        xs = jnp.take(xp, srcp, axis=0)
        xd = jnp.take(xp, dstp, axis=0)
        if li == 0:
            m2, mx = _edge_call(True, has, hbd, xs, xd, efp, wd_, wc_, be1_,
                                t(lp['e2']['W']), row(lp['e2']['b']),
                                t(lp['c1']['W']), row(lp['c1']['b']),
                                row(lp['c2']['W'][0]))
            hsum = jax.ops.segment_sum(m2, dstp, num_segments=N_NODES)
            msum = jax.ops.segment_sum(mx, dstp, num_segments=N_NODES)
            n2p = p['egnn'][1]
            h, xp, ha, hb = _node_call(
                h, hsum[None], msum[None], xp,
                t(lp['n1']['W'][:, :D]), t(lp['n1']['W'][:, D:]),
                row(lp['n1']['b']), t(lp['n2']['W']), row(lp['n2']['b']),
                row(p['gc_ln']['g']), row(p['gc_ln']['b']),
                t(n2p['e1']['W'][:, :D]), t(n2p['e1']['W'][:, D:2 * D]))
        else:
            m2 = _edge_call(False, has, hbd, xs, xd, efp, wd_, wc_, be1_,
                            t(lp['e2']['W']), row(lp['e2']['b']))
            hsum = jax.ops.segment_sum(m2, dstp, num_segments=N_NODES)
            eps = jax.random.uniform(jax.random.key(42), (N_NODES, LAT),
                                     dtype=jnp.float32)
            z, mu, lv = _node_final_call(
                h, hsum[None],
                t(lp['n1']['W'][:, :D]), t(lp['n1']['W'][:, D:]),
                row(lp['n1']['b']), t(lp['n2']['W']), row(lp['n2']['b']),
                t(p['mu']['W']), row(p['mu']['b']),
                t(p['lv']['W']), row(p['lv']['b']), eps)

    zsp = jnp.take(z, masked_edges[0].astype(jnp.int32), axis=0)
    zdp = jnp.take(z, masked_edges[1].astype(jnp.int32), axis=0)
    zsn = jnp.take(z, neg_edges[0].astype(jnp.int32), axis=0)
    zdn = jnp.take(z, neg_edges[1].astype(jnp.int32), axis=0)
    pos_out, neg_out = _dec_call(zsp, zdp, zsn, zdn,
                                 t(p['dec0']['W']), row(p['dec0']['b']),
                                 t(p['dec1']['W']), row(p['dec1']['b']))
    return (z, mu, lv, pos_out, neg_out)


# speedup vs baseline: 1.7481x; 1.7481x over previous
import functools

import jax
import jax.numpy as jnp
from jax import lax
from jax.experimental import pallas as pl
from jax.experimental.pallas import tpu as pltpu
import jax.experimental.pallas.tpu_sc as plsc

N_NODES = 10000
N_EDGES = 160000
N_MASKED = 112000
D = 128
DE = 16
LAT = 64
W = 256

NB = 1000
EB = 2048
E_P = 163840
MB = 2048

NC = 2
NS = 16
NW = NC * NS
GL = 128
SG = 64
E_SW = E_P // (NW * SG)
E_GW = E_P // (NW * GL)
M_P = 114688
M_GW = M_P // (NW * GL)
N_PAD = 10240
RT = N_PAD // NS
RZ = 128


def _ln(x, g, b, eps=1e-5):
    mu = x.mean(-1, keepdims=True)
    var = ((x - mu) ** 2).mean(-1, keepdims=True)
    return (x - mu) / jnp.sqrt(var + eps) * g + b


def _silu(x):
    return x * jax.nn.sigmoid(x)


def _dot(a, b):
    return jnp.dot(a, b, preferred_element_type=jnp.float32)


def _pack(proj, xp):
    pad = jnp.zeros((proj.shape[0], W - D - DE), jnp.float32)
    return jnp.concatenate([proj, xp, pad], axis=1)


def _embed_body(x, xp, w1, b1, g1, c1, w2, b2, g2, c2, wa, wb,
                h_ref, sa_ref, sb_ref):
    h = _dot(x[...], w1[...]) + b1[...]
    h = jax.nn.gelu(_ln(h, g1[...], c1[...]))
    h = _dot(h, w2[...]) + b2[...]
    h = jax.nn.gelu(_ln(h, g2[...], c2[...]))
    h_ref[...] = h
    sa_ref[...] = _pack(_dot(h, wa[...]), xp[...])
    sb_ref[...] = _pack(_dot(h, wb[...]), xp[...])


def _full(shape):
    return pl.BlockSpec(shape, lambda i: (0,) * len(shape))


def _rows(nb, d):
    return pl.BlockSpec((nb, d), lambda i: (i, 0))


def _embed_call(x, xp, w1, b1, g1, c1, w2, b2, g2, c2, wa, wb):
    f = pl.pallas_call(
        _embed_body,
        grid=(N_NODES // NB,),
        in_specs=[_rows(NB, D), _rows(NB, DE)]
        + [_full(a.shape) for a in (w1, b1, g1, c1, w2, b2, g2, c2, wa, wb)],
        out_specs=[_rows(NB, D), _rows(NB, W), _rows(NB, W)],
        out_shape=[jax.ShapeDtypeStruct((N_NODES, D), jnp.float32),
                   jax.ShapeDtypeStruct((N_NODES, W), jnp.float32),
                   jax.ShapeDtypeStruct((N_NODES, W), jnp.float32)],
    )
    return f(x, xp, w1, b1, g1, c1, w2, b2, g2, c2, wa, wb)


def _make_edge_body(with_coord):
    if with_coord:
        def body(ga, gb, ef, wd, wc, be1, w2, b2, wc1, bc1, wc2,
                 m2_ref, mx_ref):
            row0 = pl.program_id(0) * EB
            rid = row0 + jax.lax.broadcasted_iota(jnp.int32, (EB, 1), 0)
            valid = (rid < N_EDGES).astype(jnp.float32)
            hsum = ga[:, :D] + gb[:, :D]
            dx = ga[:, D:D + DE] - gb[:, D:D + DE]
            radial = jnp.sum(dx * dx, axis=-1, keepdims=True)
            m1 = _silu(hsum + radial * wc[...]
                       + _dot(ef[...], wd[...]) + be1[...])
            m2 = _silu(_dot(m1, w2[...]) + b2[...])
            t = _silu(_dot(m2, wc1[...]) + bc1[...])
            cw = jnp.sum(t * wc2[...], axis=-1, keepdims=True)
            col = jax.lax.broadcasted_iota(jnp.int32, (1, DE), 1)
            e3 = jnp.where(col == 3, 1.0, 0.0)
            m2_ref[...] = m2 * valid
            mx_ref[...] = (cw * dx + e3) * valid
        return body
    else:
        def body(ga, gb, ef, wd, wc, be1, w2, b2, m2_ref):
            row0 = pl.program_id(0) * EB
            rid = row0 + jax.lax.broadcasted_iota(jnp.int32, (EB, 1), 0)
            valid = (rid < N_EDGES).astype(jnp.float32)
            hsum = ga[:, :D] + gb[:, :D]
            dx = ga[:, D:D + DE] - gb[:, D:D + DE]
            radial = jnp.sum(dx * dx, axis=-1, keepdims=True)
            m1 = _silu(hsum + radial * wc[...]
                       + _dot(ef[...], wd[...]) + be1[...])
            m2 = _silu(_dot(m1, w2[...]) + b2[...])
            m2_ref[...] = m2 * valid
        return body


def _edge_call(with_coord, ga, gb, ef, wd, wc, be1, w2, b2,
               wc1=None, bc1=None, wc2=None):
    args = [ga, gb, ef, wd, wc, be1, w2, b2]
    n_out = 1
    if with_coord:
        args += [wc1, bc1, wc2]
        n_out = 2
    in_specs = ([_rows(EB, W), _rows(EB, W), _rows(EB, DE)]
                + [_full(a.shape) for a in args[3:]])
    out_specs = [_rows(EB, D)] + ([_rows(EB, DE)] if with_coord else [])
    out_shape = [jax.ShapeDtypeStruct((E_P, D), jnp.float32)] + (
        [jax.ShapeDtypeStruct((E_P, DE), jnp.float32)] if with_coord else [])
    f = pl.pallas_call(
        _make_edge_body(with_coord),
        grid=(E_P // EB,),
        in_specs=in_specs,
        out_specs=out_specs if n_out > 1 else out_specs[0],
        out_shape=out_shape if n_out > 1 else out_shape[0],
    )
    return f(*args)


def _make_node_body(n_partial, final):
    if not final:
        def body(h, hs, ms, xp, wnh, wnn, bn1, wn2, bn2, g, c, wa, wb,
                 h_ref, sa_ref, sb_ref):
            hsum = hs[0]
            msum = ms[0]
            for k in range(1, n_partial):
                hsum = hsum + hs[k]
                msum = msum + ms[k]
            cnt = jnp.maximum(msum[:, 3:4], 1.0)
            hn = _silu(_dot(h[...], wnh[...]) + _dot(hsum, wnn[...]) + bn1[...])
            hh = _dot(hn, wn2[...]) + bn2[...]
            h1 = _ln(hh, g[...], c[...])
            col = jax.lax.broadcasted_iota(jnp.int32, (1, DE), 1)
            m3 = jnp.where(col < 3, 1.0, 0.0)
            x2 = xp[...] + (msum * m3) / cnt
            h_ref[...] = h1
            sa_ref[...] = _pack(_dot(h1, wa[...]), x2)
            sb_ref[...] = _pack(_dot(h1, wb[...]), x2)
        return body
    else:
        def body(h, hs, wnh, wnn, bn1, wn2, bn2, wmu, bmu, wlv, blv, eps,
                 z_ref, mu_ref, lv_ref):
            hsum = hs[0]
            for k in range(1, n_partial):
                hsum = hsum + hs[k]
            hn = _silu(_dot(h[...], wnh[...]) + _dot(hsum, wnn[...]) + bn1[...])
            h2 = _dot(hn, wn2[...]) + bn2[...]
            mu = _dot(h2, wmu[...]) + bmu[...]
            lv = _dot(h2, wlv[...]) + blv[...]
            z_ref[...] = eps[...] * jnp.exp(lv) + mu
            mu_ref[...] = mu
            lv_ref[...] = lv
        return body


def _part(nb, d, n_partial):
    return pl.BlockSpec((n_partial, nb, d), lambda i: (0, i, 0))


def _node_call(h, hs, ms, xp, wnh, wnn, bn1, wn2, bn2, g, c, wa, wb):
    np_ = hs.shape[0]
    f = pl.pallas_call(
        _make_node_body(np_, final=False),
        grid=(N_NODES // NB,),
        in_specs=[_rows(NB, D), _part(NB, D, np_), _part(NB, DE, np_),
                  _rows(NB, DE)] + [_full(a.shape) for a in
                                    (wnh, wnn, bn1, wn2, bn2, g, c, wa, wb)],
        out_specs=[_rows(NB, D), _rows(NB, W), _rows(NB, W)],
        out_shape=[jax.ShapeDtypeStruct((N_NODES, D), jnp.float32),
                   jax.ShapeDtypeStruct((N_NODES, W), jnp.float32),
                   jax.ShapeDtypeStruct((N_NODES, W), jnp.float32)],
    )
    return f(h, hs, ms, xp, wnh, wnn, bn1, wn2, bn2, g, c, wa, wb)


def _node_final_call(h, hs, wnh, wnn, bn1, wn2, bn2, wmu, bmu, wlv, blv, eps):
    np_ = hs.shape[0]
    f = pl.pallas_call(
        _make_node_body(np_, final=True),
        grid=(N_NODES // NB,),
        in_specs=[_rows(NB, D), _part(NB, D, np_)]
        + [_full(a.shape) for a in (wnh, wnn, bn1, wn2, bn2, wmu, bmu, wlv, blv)]
        + [_rows(NB, LAT)],
        out_specs=[_rows(NB, LAT)] * 3,
        out_shape=[jax.ShapeDtypeStruct((N_NODES, LAT), jnp.float32)] * 3,
    )
    return f(h, hs, wnh, wnn, bn1, wn2, bn2, wmu, bmu, wlv, blv, eps)


def _dec_body(zsp, zdp, zsn, zdn, w0, b0, w1, b1, pos_ref, neg_ref):
    xp = zsp[:, :LAT] * zdp[:, :LAT]
    xn = zsn[:, :LAT] * zdn[:, :LAT]
    ap = jax.nn.relu(_dot(xp, w0[...]) + b0[...])
    an = jax.nn.relu(_dot(xn, w0[...]) + b0[...])
    pos_ref[...] = jax.nn.sigmoid(_dot(ap, w1[...]) + b1[...])
    neg_ref[...] = jax.nn.sigmoid(_dot(an, w1[...]) + b1[...])


def _dec_call(zsp, zdp, zsn, zdn, w0, b0, w1, b1):
    f = pl.pallas_call(
        _dec_body,
        grid=(pl.cdiv(N_MASKED, MB),),
        in_specs=[_rows(MB, D)] * 4 + [_full(a.shape) for a in (w0, b0, w1, b1)],
        out_specs=[_rows(MB, D)] * 2,
        out_shape=[jax.ShapeDtypeStruct((N_MASKED, D), jnp.float32)] * 2,
    )
    return f(zsp, zdp, zsn, zdn, w0, b0, w1, b1)


def _sc_mesh():
    return plsc.VectorSubcoreMesh(core_axis_name="c", subcore_axis_name="s")


def _sc_gather_layer(sa, sb, src_g, dst_g):
    @functools.partial(
        pl.kernel, mesh=_sc_mesh(),
        out_type=[jax.ShapeDtypeStruct((E_P, W), jnp.float32),
                  jax.ShapeDtypeStruct((E_P, W), jnp.float32)],
        scratch_types=[
            pltpu.VMEM((E_SW, SG), jnp.int32),
            pltpu.VMEM((E_SW, SG), jnp.int32),
            pltpu.VMEM((SG, W), jnp.float32), pltpu.VMEM((SG, W), jnp.float32),
            pltpu.VMEM((SG, W), jnp.float32), pltpu.VMEM((SG, W), jnp.float32),
            pltpu.SemaphoreType.DMA, pltpu.SemaphoreType.DMA,
        ],
    )
    def k(sa_h, sb_h, sg_h, dg_h, oa, ob,
          si, di, a0, b0, a1, b1, gsem, ssem):
        wid = lax.axis_index("c") * NS + lax.axis_index("s")
        g0 = wid * E_SW
        pltpu.sync_copy(sg_h.at[wid], si)
        pltpu.sync_copy(dg_h.at[wid], di)

        def body(jj, _):
            j0 = 2 * jj
            j1 = j0 + 1
            cs = []
            for (j, ba, bb) in ((j0, a0, b0), (j1, a1, b1)):
                cs.append(pltpu.async_copy(sa_h.at[si.at[j]], ba, gsem))
                cs.append(pltpu.async_copy(sb_h.at[di.at[j]], bb, gsem))
            for c in cs:
                c.wait()
            ws = []
            for (j, ba, bb) in ((j0, a0, b0), (j1, a1, b1)):
                base = (g0 + j) * SG
                ws.append(pltpu.async_copy(ba, oa.at[pl.ds(base, SG)], ssem))
                ws.append(pltpu.async_copy(bb, ob.at[pl.ds(base, SG)], ssem))
            for w in ws:
                w.wait()
            return 0
        lax.fori_loop(0, E_SW // 2, body, 0)
    return k(sa, sb, src_g, dst_g)


def _sc_gather_dec(zp, g0_, g1_, g2_, g3_):
    @functools.partial(
        pl.kernel, mesh=_sc_mesh(),
        out_type=[jax.ShapeDtypeStruct((M_P, D), jnp.float32)] * 4,
        scratch_types=[pltpu.VMEM((M_GW, GL), jnp.int32)] * 4
        + [pltpu.VMEM((GL, D), jnp.float32)] * 4
        + [pltpu.SemaphoreType.DMA, pltpu.SemaphoreType.DMA],
    )
    def k(z_h, i0h, i1h, i2h, i3h, o0, o1, o2, o3,
          v0, v1, v2, v3, b0, b1, b2, b3, gsem, ssem):
        wid = lax.axis_index("c") * NS + lax.axis_index("s")
        g0 = wid * M_GW
        for ih, iv in ((i0h, v0), (i1h, v1), (i2h, v2), (i3h, v3)):
            pltpu.sync_copy(ih.at[wid], iv)

        def body(j, _):
            base = (g0 + j) * GL
            cs = [pltpu.async_copy(z_h.at[iv.at[j]], bb, gsem)
                  for iv, bb in ((v0, b0), (v1, b1), (v2, b2), (v3, b3))]
            for c in cs:
                c.wait()
            ws = [pltpu.async_copy(bb, oo.at[pl.ds(base, GL)], ssem)
                  for bb, oo in ((b0, o0), (b1, o1), (b2, o2), (b3, o3))]
            for w in ws:
                w.wait()
            return 0
        lax.fori_loop(0, M_GW, body, 0)
    return k(zp, g0_, g1_, g2_, g3_)


def _zero_vmem(ref, rows, cols):
    def zr(i, _):
        def zc(c, __):
            ref[i, pl.ds(c * 16, 16)] = jnp.zeros((16,), jnp.float32)
            return 0
        lax.fori_loop(0, cols // 16, zc, 0)
        return 0
    lax.fori_loop(0, rows, zr, 0)


def _make_scatter(with_mx):
    out_type = [jax.ShapeDtypeStruct((NC, N_PAD, D), jnp.float32)]
    scratch = [pltpu.VMEM((E_GW, GL), jnp.int32),
               pltpu.VMEM((GL, D), jnp.float32),
               pltpu.VMEM_SHARED((N_PAD, D), jnp.float32)]
    if with_mx:
        out_type.append(jax.ShapeDtypeStruct((NC, N_PAD, DE), jnp.float32))
        scratch += [pltpu.VMEM((GL, DE), jnp.float32),
                    pltpu.VMEM_SHARED((N_PAD, DE), jnp.float32)]

    def body_mx(m2_h, mx_h, dg_h, hpart, mpart,
                idx, buf, acc_h, bufx, acc_x):
        _scatter_common(m2_h, dg_h, hpart, idx, buf, acc_h,
                        mx_h, mpart, bufx, acc_x)

    def body_h(m2_h, dg_h, hpart, idx, buf, acc_h):
        _scatter_common(m2_h, dg_h, hpart, idx, buf, acc_h,
                        None, None, None, None)

    body = body_mx if with_mx else body_h
    return functools.partial(pl.kernel, mesh=_sc_mesh(), out_type=out_type,
                             scratch_types=scratch)(body)


def _scatter_common(m2_h, dg_h, hpart, idx, buf, acc_h,
                    mx_h, mpart, bufx, acc_x):
    cid = lax.axis_index("c")
    sid = lax.axis_index("s")
    wid = cid * NS + sid
    _zero_vmem(buf, RZ, D)
    if mx_h is not None:
        _zero_vmem(bufx, RZ, DE)
    for k5 in range(RT // RZ):
        rows = pl.ds(sid * RT + k5 * RZ, RZ)
        pltpu.sync_copy(buf, acc_h.at[rows])
        if mx_h is not None:
            pltpu.sync_copy(bufx, acc_x.at[rows])
    plsc.subcore_barrier()
    pltpu.sync_copy(dg_h.at[wid], idx)

    def body(j, _):
        base = (wid * E_GW + j) * GL
        pltpu.sync_copy(m2_h.at[pl.ds(base, GL)], buf)
        pltpu.sync_copy(buf, acc_h.at[idx.at[j]], add=True)
        if mx_h is not None:
            pltpu.sync_copy(mx_h.at[pl.ds(base, GL)], bufx)
            pltpu.sync_copy(bufx, acc_x.at[idx.at[j]], add=True)
        return 0
    lax.fori_loop(0, E_GW, body, 0)
    plsc.subcore_barrier()
    for k5 in range(RT // RZ):
        rows = pl.ds(sid * RT + k5 * RZ, RZ)
        pltpu.sync_copy(acc_h.at[rows], hpart.at[cid, rows])
        if mx_h is not None:
            pltpu.sync_copy(acc_x.at[rows], mpart.at[cid, rows])


@functools.lru_cache(maxsize=None)
def _scatter_kernel(with_mx):
    return _make_scatter(with_mx)


def _scatter_mx(m2, mx, dg):
    return _scatter_kernel(True)(m2, mx, dg)


def _scatter_h(m2, dg):
    return _scatter_kernel(False)(m2, dg)


def kernel(node_feat, xyz, edge_feat, edge_index, masked_edges, neg_edges,
           params):
    p = params
    t = lambda w: w.T
    row = lambda v: v.reshape(1, -1)

    src = edge_index[0].astype(jnp.int32)
    dst = edge_index[1].astype(jnp.int32)
    pad = E_P - N_EDGES
    srcp = jnp.concatenate([src, jnp.zeros((pad,), jnp.int32)])
    dstp = jnp.concatenate([dst, jnp.zeros((pad,), jnp.int32)])
    srcg = srcp.reshape(NW, E_SW, SG)
    dstg = dstp.reshape(NW, E_SW, SG)
    dstg_sc = dstp.reshape(NW, E_GW, GL)
    efp = jnp.pad(edge_feat, ((0, pad), (0, 0)))
    xp = jnp.pad(xyz, ((0, 0), (0, DE - 3)))

    mpad = M_P - N_MASKED
    def decg(e):
        return jnp.concatenate([e.astype(jnp.int32),
                                jnp.zeros((mpad,), jnp.int32)]
                               ).reshape(NW, M_GW, GL)

    e = p['emb1']; e2 = p['emb2']
    l1, l2 = p['egnn']

    def split_e1(lp):
        Wm = lp['e1']['W']
        return (row(Wm[:, 2 * D]), t(Wm[:, 2 * D + 1:]), row(lp['e1']['b']))

    h, sa, sb = _embed_call(
        node_feat, xp, t(e['W']), row(e['b']),
        row(p['emb_ln1']['g']), row(p['emb_ln1']['b']),
        t(e2['W']), row(e2['b']),
        row(p['emb_ln2']['g']), row(p['emb_ln2']['b']),
        t(l1['e1']['W'][:, :D]), t(l1['e1']['W'][:, D:2 * D]))

    for li, lp in enumerate((l1, l2)):
        wc_, wd_, be1_ = split_e1(lp)
        ga, gb = _sc_gather_layer(sa, sb, srcg, dstg)
        if li == 0:
            m2, mx = _edge_call(True, ga, gb, efp, wd_, wc_, be1_,
                                t(lp['e2']['W']), row(lp['e2']['b']),
                                t(lp['c1']['W']), row(lp['c1']['b']),
                                row(lp['c2']['W'][0]))
            hsum = jax.ops.segment_sum(m2[:N_EDGES], dst,
                                       num_segments=N_NODES)[None]
            msum = jax.ops.segment_sum(mx[:N_EDGES], dst,
                                       num_segments=N_NODES)[None]
            n2p = p['egnn'][1]
            h, sa, sb = _node_call(
                h, hsum, msum, xp,
                t(lp['n1']['W'][:, :D]), t(lp['n1']['W'][:, D:]),
                row(lp['n1']['b']), t(lp['n2']['W']), row(lp['n2']['b']),
                row(p['gc_ln']['g']), row(p['gc_ln']['b']),
                t(n2p['e1']['W'][:, :D]), t(n2p['e1']['W'][:, D:2 * D]))
        else:
            m2 = _edge_call(False, ga, gb, efp, wd_, wc_, be1_,
                            t(lp['e2']['W']), row(lp['e2']['b']))
            hsum = jax.ops.segment_sum(m2[:N_EDGES], dst,
                                       num_segments=N_NODES)[None]
            eps = jax.random.uniform(jax.random.key(42), (N_NODES, LAT),
                                     dtype=jnp.float32)
            z, mu, lv = _node_final_call(
                h, hsum,
                t(lp['n1']['W'][:, :D]), t(lp['n1']['W'][:, D:]),
                row(lp['n1']['b']), t(lp['n2']['W']), row(lp['n2']['b']),
                t(p['mu']['W']), row(p['mu']['b']),
                t(p['lv']['W']), row(p['lv']['b']), eps)

    zp = jnp.pad(z, ((0, 0), (0, D - LAT)))
    zsp, zdp, zsn, zdn = (jnp.take(zp, decg(e).reshape(-1), axis=0)
                          for e in (masked_edges[0], masked_edges[1],
                                    neg_edges[0], neg_edges[1]))
    pos_out, neg_out = _dec_call(zsp, zdp, zsn, zdn,
                                 t(p['dec0']['W']), row(p['dec0']['b']),
                                 t(p['dec1']['W']), row(p['dec1']['b']))
    return (z, mu, lv, pos_out, neg_out)


# speedup vs baseline: 2.2640x; 1.2952x over previous
import functools

import jax
import jax.numpy as jnp
from jax import lax
from jax.experimental import pallas as pl
from jax.experimental.pallas import tpu as pltpu
import jax.experimental.pallas.tpu_sc as plsc

N_NODES = 10000
N_EDGES = 160000
N_MASKED = 112000
D = 128
DE = 16
LAT = 64
W = 256

NB = 1000
EB = 2048
E_P = 163840
MB = 2048

NC = 2
NS = 16
NW = NC * NS
GL = 128
SG = 64
E_SW = E_P // (NW * SG)
E_GW = E_P // (NW * GL)
M_P = 114688
M_GW = M_P // (NW * GL)
N_PAD = 10240
RT = N_PAD // NS
RZ = 128


def _ln(x, g, b, eps=1e-5):
    mu = x.mean(-1, keepdims=True)
    var = ((x - mu) ** 2).mean(-1, keepdims=True)
    return (x - mu) / jnp.sqrt(var + eps) * g + b


def _silu(x):
    return x * jax.nn.sigmoid(x)


def _dot(a, b):
    return jnp.dot(a, b, preferred_element_type=jnp.float32)


def _pack(proj, xp):
    pad = jnp.zeros((proj.shape[0], W - D - DE), jnp.float32)
    return jnp.concatenate([proj, xp, pad], axis=1)


def _embed_body(x, xp, w1, b1, g1, c1, w2, b2, g2, c2, wa, wb,
                h_ref, sa_ref, sb_ref):
    h = _dot(x[...], w1[...]) + b1[...]
    h = jax.nn.gelu(_ln(h, g1[...], c1[...]))
    h = _dot(h, w2[...]) + b2[...]
    h = jax.nn.gelu(_ln(h, g2[...], c2[...]))
    h_ref[...] = h
    sa_ref[...] = _pack(_dot(h, wa[...]), xp[...])
    sb_ref[...] = _pack(_dot(h, wb[...]), xp[...])


def _full(shape):
    return pl.BlockSpec(shape, lambda i: (0,) * len(shape))


def _rows(nb, d):
    return pl.BlockSpec((nb, d), lambda i: (i, 0))


def _embed_call(x, xp, w1, b1, g1, c1, w2, b2, g2, c2, wa, wb):
    f = pl.pallas_call(
        _embed_body,
        grid=(N_NODES // NB,),
        in_specs=[_rows(NB, D), _rows(NB, DE)]
        + [_full(a.shape) for a in (w1, b1, g1, c1, w2, b2, g2, c2, wa, wb)],
        out_specs=[_rows(NB, D), _rows(NB, W), _rows(NB, W)],
        out_shape=[jax.ShapeDtypeStruct((N_NODES, D), jnp.float32),
                   jax.ShapeDtypeStruct((N_NODES, W), jnp.float32),
                   jax.ShapeDtypeStruct((N_NODES, W), jnp.float32)],
    )
    return f(x, xp, w1, b1, g1, c1, w2, b2, g2, c2, wa, wb)


def _make_edge_body(with_coord):
    if with_coord:
        def body(ga, gb, ef, wd, wc, be1, w2, b2, wc1, bc1, wc2,
                 m2_ref, mx_ref):
            row0 = pl.program_id(0) * EB
            rid = row0 + jax.lax.broadcasted_iota(jnp.int32, (EB, 1), 0)
            valid = (rid < N_EDGES).astype(jnp.float32)
            hsum = ga[:, :D] + gb[:, :D]
            dx = ga[:, D:D + DE] - gb[:, D:D + DE]
            radial = jnp.sum(dx * dx, axis=-1, keepdims=True)
            m1 = _silu(hsum + radial * wc[...]
                       + _dot(ef[...], wd[...]) + be1[...])
            m2 = _silu(_dot(m1, w2[...]) + b2[...])
            t = _silu(_dot(m2, wc1[...]) + bc1[...])
            cw = jnp.sum(t * wc2[...], axis=-1, keepdims=True)
            col = jax.lax.broadcasted_iota(jnp.int32, (1, DE), 1)
            e3 = jnp.where(col == 3, 1.0, 0.0)
            m2_ref[...] = m2 * valid
            mx_ref[...] = (cw * dx + e3) * valid
        return body
    else:
        def body(ga, gb, ef, wd, wc, be1, w2, b2, m2_ref):
            row0 = pl.program_id(0) * EB
            rid = row0 + jax.lax.broadcasted_iota(jnp.int32, (EB, 1), 0)
            valid = (rid < N_EDGES).astype(jnp.float32)
            hsum = ga[:, :D] + gb[:, :D]
            dx = ga[:, D:D + DE] - gb[:, D:D + DE]
            radial = jnp.sum(dx * dx, axis=-1, keepdims=True)
            m1 = _silu(hsum + radial * wc[...]
                       + _dot(ef[...], wd[...]) + be1[...])
            m2 = _silu(_dot(m1, w2[...]) + b2[...])
            m2_ref[...] = m2 * valid
        return body


def _edge_call(with_coord, ga, gb, ef, wd, wc, be1, w2, b2,
               wc1=None, bc1=None, wc2=None):
    args = [ga, gb, ef, wd, wc, be1, w2, b2]
    n_out = 1
    if with_coord:
        args += [wc1, bc1, wc2]
        n_out = 2
    in_specs = ([_rows(EB, W), _rows(EB, W), _rows(EB, DE)]
                + [_full(a.shape) for a in args[3:]])
    out_specs = [_rows(EB, D)] + ([_rows(EB, DE)] if with_coord else [])
    out_shape = [jax.ShapeDtypeStruct((E_P, D), jnp.float32)] + (
        [jax.ShapeDtypeStruct((E_P, DE), jnp.float32)] if with_coord else [])
    f = pl.pallas_call(
        _make_edge_body(with_coord),
        grid=(E_P // EB,),
        in_specs=in_specs,
        out_specs=out_specs if n_out > 1 else out_specs[0],
        out_shape=out_shape if n_out > 1 else out_shape[0],
    )
    return f(*args)


def _make_node_body(n_partial, final):
    if not final:
        def body(h, hs, ms, xp, wnh, wnn, bn1, wn2, bn2, g, c, wa, wb,
                 h_ref, sa_ref, sb_ref):
            hsum = hs[0]
            msum = ms[0]
            for k in range(1, n_partial):
                hsum = hsum + hs[k]
                msum = msum + ms[k]
            cnt = jnp.maximum(msum[:, 3:4], 1.0)
            hn = _silu(_dot(h[...], wnh[...]) + _dot(hsum, wnn[...]) + bn1[...])
            hh = _dot(hn, wn2[...]) + bn2[...]
            h1 = _ln(hh, g[...], c[...])
            col = jax.lax.broadcasted_iota(jnp.int32, (1, DE), 1)
            m3 = jnp.where(col < 3, 1.0, 0.0)
            x2 = xp[...] + (msum * m3) / cnt
            h_ref[...] = h1
            sa_ref[...] = _pack(_dot(h1, wa[...]), x2)
            sb_ref[...] = _pack(_dot(h1, wb[...]), x2)
        return body
    else:
        def body(h, hs, wnh, wnn, bn1, wn2, bn2, wmu, bmu, wlv, blv, eps,
                 z_ref, mu_ref, lv_ref):
            hsum = hs[0]
            for k in range(1, n_partial):
                hsum = hsum + hs[k]
            hn = _silu(_dot(h[...], wnh[...]) + _dot(hsum, wnn[...]) + bn1[...])
            h2 = _dot(hn, wn2[...]) + bn2[...]
            mu = _dot(h2, wmu[...]) + bmu[...]
            lv = _dot(h2, wlv[...]) + blv[...]
            z_ref[...] = eps[...] * jnp.exp(lv) + mu
            mu_ref[...] = mu
            lv_ref[...] = lv
        return body


def _part(nb, d, n_partial):
    return pl.BlockSpec((n_partial, nb, d), lambda i: (0, i, 0))


def _node_call(h, hs, ms, xp, wnh, wnn, bn1, wn2, bn2, g, c, wa, wb):
    np_ = hs.shape[0]
    f = pl.pallas_call(
        _make_node_body(np_, final=False),
        grid=(N_NODES // NB,),
        in_specs=[_rows(NB, D), _part(NB, D, np_), _part(NB, DE, np_),
                  _rows(NB, DE)] + [_full(a.shape) for a in
                                    (wnh, wnn, bn1, wn2, bn2, g, c, wa, wb)],
        out_specs=[_rows(NB, D), _rows(NB, W), _rows(NB, W)],
        out_shape=[jax.ShapeDtypeStruct((N_NODES, D), jnp.float32),
                   jax.ShapeDtypeStruct((N_NODES, W), jnp.float32),
                   jax.ShapeDtypeStruct((N_NODES, W), jnp.float32)],
    )
    return f(h, hs, ms, xp, wnh, wnn, bn1, wn2, bn2, g, c, wa, wb)


def _node_final_call(h, hs, wnh, wnn, bn1, wn2, bn2, wmu, bmu, wlv, blv, eps):
    np_ = hs.shape[0]
    f = pl.pallas_call(
        _make_node_body(np_, final=True),
        grid=(N_NODES // NB,),
        in_specs=[_rows(NB, D), _part(NB, D, np_)]
        + [_full(a.shape) for a in (wnh, wnn, bn1, wn2, bn2, wmu, bmu, wlv, blv)]
        + [_rows(NB, LAT)],
        out_specs=[_rows(NB, LAT)] * 3,
        out_shape=[jax.ShapeDtypeStruct((N_NODES, LAT), jnp.float32)] * 3,
    )
    return f(h, hs, wnh, wnn, bn1, wn2, bn2, wmu, bmu, wlv, blv, eps)


def _dec_body(zsp, zdp, zsn, zdn, w0, b0, w1, b1, pos_ref, neg_ref):
    xp = zsp[:, :LAT] * zdp[:, :LAT]
    xn = zsn[:, :LAT] * zdn[:, :LAT]
    ap = jax.nn.relu(_dot(xp, w0[...]) + b0[...])
    an = jax.nn.relu(_dot(xn, w0[...]) + b0[...])
    pos_ref[...] = jax.nn.sigmoid(_dot(ap, w1[...]) + b1[...])
    neg_ref[...] = jax.nn.sigmoid(_dot(an, w1[...]) + b1[...])


def _dec_call(zsp, zdp, zsn, zdn, w0, b0, w1, b1):
    f = pl.pallas_call(
        _dec_body,
        grid=(pl.cdiv(N_MASKED, MB),),
        in_specs=[_rows(MB, D)] * 4 + [_full(a.shape) for a in (w0, b0, w1, b1)],
        out_specs=[_rows(MB, D)] * 2,
        out_shape=[jax.ShapeDtypeStruct((N_MASKED, D), jnp.float32)] * 2,
    )
    return f(zsp, zdp, zsn, zdn, w0, b0, w1, b1)


def _sc_mesh():
    return plsc.VectorSubcoreMesh(core_axis_name="c", subcore_axis_name="s")


def _sc_gather_layer(sa, sb, src_g, dst_g):
    @functools.partial(
        pl.kernel, mesh=_sc_mesh(),
        out_type=[jax.ShapeDtypeStruct((E_P, W), jnp.float32),
                  jax.ShapeDtypeStruct((E_P, W), jnp.float32)],
        scratch_types=[
            pltpu.VMEM((E_SW, SG), jnp.int32),
            pltpu.VMEM((E_SW, SG), jnp.int32),
            pltpu.VMEM((SG, W), jnp.float32), pltpu.VMEM((SG, W), jnp.float32),
            pltpu.VMEM((SG, W), jnp.float32), pltpu.VMEM((SG, W), jnp.float32),
            pltpu.SemaphoreType.DMA, pltpu.SemaphoreType.DMA,
        ],
    )
    def k(sa_h, sb_h, sg_h, dg_h, oa, ob,
          si, di, a0, b0, a1, b1, gsem, ssem):
        wid = lax.axis_index("c") * NS + lax.axis_index("s")
        g0 = wid * E_SW
        pltpu.sync_copy(sg_h.at[wid], si)
        pltpu.sync_copy(dg_h.at[wid], di)

        def body(jj, _):
            j0 = 2 * jj
            j1 = j0 + 1
            cs = []
            for (j, ba, bb) in ((j0, a0, b0), (j1, a1, b1)):
                cs.append(pltpu.async_copy(sa_h.at[si.at[j]], ba, gsem))
                cs.append(pltpu.async_copy(sb_h.at[di.at[j]], bb, gsem))
            for c in cs:
                c.wait()
            ws = []
            for (j, ba, bb) in ((j0, a0, b0), (j1, a1, b1)):
                base = (g0 + j) * SG
                ws.append(pltpu.async_copy(ba, oa.at[pl.ds(base, SG)], ssem))
                ws.append(pltpu.async_copy(bb, ob.at[pl.ds(base, SG)], ssem))
            for w in ws:
                w.wait()
            return 0
        lax.fori_loop(0, E_SW // 2, body, 0)
    return k(sa, sb, src_g, dst_g)


def _sc_gather_dec(zp, g0_, g1_, g2_, g3_):
    @functools.partial(
        pl.kernel, mesh=_sc_mesh(),
        out_type=[jax.ShapeDtypeStruct((M_P, D), jnp.float32)] * 4,
        scratch_types=[pltpu.VMEM((M_GW, GL), jnp.int32)] * 4
        + [pltpu.VMEM((GL, D), jnp.float32)] * 4
        + [pltpu.SemaphoreType.DMA, pltpu.SemaphoreType.DMA],
    )
    def k(z_h, i0h, i1h, i2h, i3h, o0, o1, o2, o3,
          v0, v1, v2, v3, b0, b1, b2, b3, gsem, ssem):
        wid = lax.axis_index("c") * NS + lax.axis_index("s")
        g0 = wid * M_GW
        for ih, iv in ((i0h, v0), (i1h, v1), (i2h, v2), (i3h, v3)):
            pltpu.sync_copy(ih.at[wid], iv)

        def body(j, _):
            base = (g0 + j) * GL
            cs = [pltpu.async_copy(z_h.at[iv.at[j]], bb, gsem)
                  for iv, bb in ((v0, b0), (v1, b1), (v2, b2), (v3, b3))]
            for c in cs:
                c.wait()
            ws = [pltpu.async_copy(bb, oo.at[pl.ds(base, GL)], ssem)
                  for bb, oo in ((b0, o0), (b1, o1), (b2, o2), (b3, o3))]
            for w in ws:
                w.wait()
            return 0
        lax.fori_loop(0, M_GW, body, 0)
    return k(zp, g0_, g1_, g2_, g3_)


def _zero_vmem(ref, rows, cols):
    def zr(i, _):
        def zc(c, __):
            ref[i, pl.ds(c * 16, 16)] = jnp.zeros((16,), jnp.float32)
            return 0
        lax.fori_loop(0, cols // 16, zc, 0)
        return 0
    lax.fori_loop(0, rows, zr, 0)


def _make_scatter(with_mx):
    out_type = [jax.ShapeDtypeStruct((NC, N_PAD, D), jnp.float32)]
    scratch = [pltpu.VMEM((E_GW, GL), jnp.int32),
               pltpu.VMEM((GL, D), jnp.float32),
               pltpu.VMEM_SHARED((N_PAD, D), jnp.float32)]
    if with_mx:
        out_type.append(jax.ShapeDtypeStruct((NC, N_PAD, DE), jnp.float32))
        scratch += [pltpu.VMEM((GL, DE), jnp.float32),
                    pltpu.VMEM_SHARED((N_PAD, DE), jnp.float32)]

    def body_mx(m2_h, mx_h, dg_h, hpart, mpart,
                idx, buf, acc_h, bufx, acc_x):
        _scatter_common(m2_h, dg_h, hpart, idx, buf, acc_h,
                        mx_h, mpart, bufx, acc_x)

    def body_h(m2_h, dg_h, hpart, idx, buf, acc_h):
        _scatter_common(m2_h, dg_h, hpart, idx, buf, acc_h,
                        None, None, None, None)

    body = body_mx if with_mx else body_h
    return functools.partial(pl.kernel, mesh=_sc_mesh(), out_type=out_type,
                             scratch_types=scratch)(body)


def _scatter_common(m2_h, dg_h, hpart, idx, buf, acc_h,
                    mx_h, mpart, bufx, acc_x):
    cid = lax.axis_index("c")
    sid = lax.axis_index("s")
    wid = cid * NS + sid
    _zero_vmem(buf, RZ, D)
    if mx_h is not None:
        _zero_vmem(bufx, RZ, DE)
    for k5 in range(RT // RZ):
        rows = pl.ds(sid * RT + k5 * RZ, RZ)
        pltpu.sync_copy(buf, acc_h.at[rows])
        if mx_h is not None:
            pltpu.sync_copy(bufx, acc_x.at[rows])
    plsc.subcore_barrier()
    pltpu.sync_copy(dg_h.at[wid], idx)

    def body(j, _):
        base = (wid * E_GW + j) * GL
        pltpu.sync_copy(m2_h.at[pl.ds(base, GL)], buf)
        pltpu.sync_copy(buf, acc_h.at[idx.at[j]], add=True)
        if mx_h is not None:
            pltpu.sync_copy(mx_h.at[pl.ds(base, GL)], bufx)
            pltpu.sync_copy(bufx, acc_x.at[idx.at[j]], add=True)
        return 0
    lax.fori_loop(0, E_GW, body, 0)
    plsc.subcore_barrier()
    for k5 in range(RT // RZ):
        rows = pl.ds(sid * RT + k5 * RZ, RZ)
        pltpu.sync_copy(acc_h.at[rows], hpart.at[cid, rows])
        if mx_h is not None:
            pltpu.sync_copy(acc_x.at[rows], mpart.at[cid, rows])


@functools.lru_cache(maxsize=None)
def _scatter_kernel(with_mx):
    return _make_scatter(with_mx)


def _scatter_mx(m2, mx, dg):
    return _scatter_kernel(True)(m2, mx, dg)


def _scatter_h(m2, dg):
    return _scatter_kernel(False)(m2, dg)


def kernel(node_feat, xyz, edge_feat, edge_index, masked_edges, neg_edges,
           params):
    p = params
    t = lambda w: w.T
    row = lambda v: v.reshape(1, -1)

    src = edge_index[0].astype(jnp.int32)
    dst = edge_index[1].astype(jnp.int32)
    pad = E_P - N_EDGES
    srcp = jnp.concatenate([src, jnp.zeros((pad,), jnp.int32)])
    dstp = jnp.concatenate([dst, jnp.zeros((pad,), jnp.int32)])
    srcg = srcp.reshape(NW, E_SW, SG)
    dstg = dstp.reshape(NW, E_SW, SG)
    dstg_sc = dstp.reshape(NW, E_GW, GL)
    efp = jnp.pad(edge_feat, ((0, pad), (0, 0)))
    xp = jnp.pad(xyz, ((0, 0), (0, DE - 3)))

    mpad = M_P - N_MASKED
    def decg(e):
        return jnp.concatenate([e.astype(jnp.int32),
                                jnp.zeros((mpad,), jnp.int32)]
                               ).reshape(NW, M_GW, GL)

    e = p['emb1']; e2 = p['emb2']
    l1, l2 = p['egnn']

    def split_e1(lp):
        Wm = lp['e1']['W']
        return (row(Wm[:, 2 * D]), t(Wm[:, 2 * D + 1:]), row(lp['e1']['b']))

    h, sa, sb = _embed_call(
        node_feat, xp, t(e['W']), row(e['b']),
        row(p['emb_ln1']['g']), row(p['emb_ln1']['b']),
        t(e2['W']), row(e2['b']),
        row(p['emb_ln2']['g']), row(p['emb_ln2']['b']),
        t(l1['e1']['W'][:, :D]), t(l1['e1']['W'][:, D:2 * D]))

    for li, lp in enumerate((l1, l2)):
        wc_, wd_, be1_ = split_e1(lp)
        ga, gb = _sc_gather_layer(sa, sb, srcg, dstg)
        if li == 0:
            m2, mx = _edge_call(True, ga, gb, efp, wd_, wc_, be1_,
                                t(lp['e2']['W']), row(lp['e2']['b']),
                                t(lp['c1']['W']), row(lp['c1']['b']),
                                row(lp['c2']['W'][0]))
            hsum = jax.ops.segment_sum(m2[:N_EDGES], dst,
                                       num_segments=N_NODES)[None]
            msum = jax.ops.segment_sum(mx[:N_EDGES], dst,
                                       num_segments=N_NODES)[None]
            n2p = p['egnn'][1]
            h, sa, sb = _node_call(
                h, hsum, msum, xp,
                t(lp['n1']['W'][:, :D]), t(lp['n1']['W'][:, D:]),
                row(lp['n1']['b']), t(lp['n2']['W']), row(lp['n2']['b']),
                row(p['gc_ln']['g']), row(p['gc_ln']['b']),
                t(n2p['e1']['W'][:, :D]), t(n2p['e1']['W'][:, D:2 * D]))
        else:
            m2 = _edge_call(False, ga, gb, efp, wd_, wc_, be1_,
                            t(lp['e2']['W']), row(lp['e2']['b']))
            hsum = jax.ops.segment_sum(m2[:N_EDGES], dst,
                                       num_segments=N_NODES)[None]
            eps = jax.random.uniform(jax.random.key(42), (N_NODES, LAT),
                                     dtype=jnp.float32)
            z, mu, lv = _node_final_call(
                h, hsum,
                t(lp['n1']['W'][:, :D]), t(lp['n1']['W'][:, D:]),
                row(lp['n1']['b']), t(lp['n2']['W']), row(lp['n2']['b']),
                t(p['mu']['W']), row(p['mu']['b']),
                t(p['lv']['W']), row(p['lv']['b']), eps)

    zp = jnp.pad(z, ((0, 0), (0, D - LAT)))
    zsp, zdp, zsn, zdn = _sc_gather_dec(
        zp, decg(masked_edges[0]), decg(masked_edges[1]),
        decg(neg_edges[0]), decg(neg_edges[1]))
    pos_out, neg_out = _dec_call(zsp, zdp, zsn, zdn,
                                 t(p['dec0']['W']), row(p['dec0']['b']),
                                 t(p['dec1']['W']), row(p['dec1']['b']))
    return (z, mu, lv, pos_out, neg_out)


# speedup vs baseline: 2.4071x; 1.0632x over previous
import functools

import jax
import jax.numpy as jnp
from jax import lax
from jax.experimental import pallas as pl
from jax.experimental.pallas import tpu as pltpu
import jax.experimental.pallas.tpu_sc as plsc

N_NODES = 10000
N_EDGES = 160000
N_MASKED = 112000
D = 128
DE = 16
LAT = 64
W = 128

NB = 1000
EB = 2048
E_P = 163840
MB = 2048

NC = 2
NS = 16
NW = NC * NS
GL = 128
SG = 64
E_SW = E_P // (NW * SG)
E_GW = E_P // (NW * GL)
M_P = 114688
M_GW = M_P // (NW * GL)
N_PAD = 10240
RT = N_PAD // NS
RZ = 128


def _ln(x, g, b, eps=1e-5):
    mu = x.mean(-1, keepdims=True)
    var = ((x - mu) ** 2).mean(-1, keepdims=True)
    return (x - mu) / jnp.sqrt(var + eps) * g + b


def _silu(x):
    return x * jax.nn.sigmoid(x)


def _dot(a, b):
    return jnp.dot(a, b, preferred_element_type=jnp.float32)


def _pack(proj, xp):
    aux = jnp.concatenate(
        [xp, jnp.zeros((proj.shape[0], D - DE), jnp.float32)], axis=1)
    pb = lax.bitcast_convert_type(proj, jnp.int32)
    ab = lax.bitcast_convert_type(aux, jnp.int32)
    half = jnp.int32(0x8000)
    hi = jnp.bitwise_and(ab + half, jnp.int32(-65536))
    lo = lax.shift_right_logical(pb + half, 16)
    return jnp.bitwise_or(hi, lo)


def _unpack(g):
    proj = lax.bitcast_convert_type(jnp.left_shift(g, 16), jnp.float32)
    aux = lax.bitcast_convert_type(
        jnp.bitwise_and(g, jnp.int32(-65536)), jnp.float32)
    return proj, aux[:, :DE]


def _embed_body(x, xp, w1, b1, g1, c1, w2, b2, g2, c2, wa, wb,
                h_ref, sa_ref, sb_ref):
    h = _dot(x[...], w1[...]) + b1[...]
    h = jax.nn.gelu(_ln(h, g1[...], c1[...]))
    h = _dot(h, w2[...]) + b2[...]
    h = jax.nn.gelu(_ln(h, g2[...], c2[...]))
    h_ref[...] = h
    sa_ref[...] = _pack(_dot(h, wa[...]), xp[...])
    sb_ref[...] = _pack(_dot(h, wb[...]), xp[...])


def _full(shape):
    return pl.BlockSpec(shape, lambda i: (0,) * len(shape))


def _rows(nb, d):
    return pl.BlockSpec((nb, d), lambda i: (i, 0))


def _embed_call(x, xp, w1, b1, g1, c1, w2, b2, g2, c2, wa, wb):
    f = pl.pallas_call(
        _embed_body,
        grid=(N_NODES // NB,),
        in_specs=[_rows(NB, D), _rows(NB, DE)]
        + [_full(a.shape) for a in (w1, b1, g1, c1, w2, b2, g2, c2, wa, wb)],
        out_specs=[_rows(NB, D), _rows(NB, W), _rows(NB, W)],
        out_shape=[jax.ShapeDtypeStruct((N_NODES, D), jnp.float32),
                   jax.ShapeDtypeStruct((N_NODES, W), jnp.int32),
                   jax.ShapeDtypeStruct((N_NODES, W), jnp.int32)],
    )
    return f(x, xp, w1, b1, g1, c1, w2, b2, g2, c2, wa, wb)


def _make_edge_body(with_coord):
    if with_coord:
        def body(ga, gb, ef, wd, wc, be1, w2, b2, wc1, bc1, wc2,
                 m2_ref, mx_ref):
            row0 = pl.program_id(0) * EB
            rid = row0 + jax.lax.broadcasted_iota(jnp.int32, (EB, 1), 0)
            valid = (rid < N_EDGES).astype(jnp.float32)
            pa, xa = _unpack(ga[...])
            pb, xb = _unpack(gb[...])
            hsum = pa + pb
            dx = xa - xb
            radial = jnp.sum(dx * dx, axis=-1, keepdims=True)
            m1 = _silu(hsum + radial * wc[...]
                       + _dot(ef[...], wd[...]) + be1[...])
            m2 = _silu(_dot(m1, w2[...]) + b2[...])
            t = _silu(_dot(m2, wc1[...]) + bc1[...])
            cw = jnp.sum(t * wc2[...], axis=-1, keepdims=True)
            col = jax.lax.broadcasted_iota(jnp.int32, (1, DE), 1)
            e3 = jnp.where(col == 3, 1.0, 0.0)
            m2_ref[...] = m2 * valid
            mx_ref[...] = (cw * dx + e3) * valid
        return body
    else:
        def body(ga, gb, ef, wd, wc, be1, w2, b2, m2_ref):
            row0 = pl.program_id(0) * EB
            rid = row0 + jax.lax.broadcasted_iota(jnp.int32, (EB, 1), 0)
            valid = (rid < N_EDGES).astype(jnp.float32)
            pa, xa = _unpack(ga[...])
            pb, xb = _unpack(gb[...])
            hsum = pa + pb
            dx = xa - xb
            radial = jnp.sum(dx * dx, axis=-1, keepdims=True)
            m1 = _silu(hsum + radial * wc[...]
                       + _dot(ef[...], wd[...]) + be1[...])
            m2 = _silu(_dot(m1, w2[...]) + b2[...])
            m2_ref[...] = m2 * valid
        return body


def _edge_call(with_coord, ga, gb, ef, wd, wc, be1, w2, b2,
               wc1=None, bc1=None, wc2=None):
    args = [ga, gb, ef, wd, wc, be1, w2, b2]
    n_out = 1
    if with_coord:
        args += [wc1, bc1, wc2]
        n_out = 2
    in_specs = ([_rows(EB, W), _rows(EB, W), _rows(EB, DE)]
                + [_full(a.shape) for a in args[3:]])
    out_specs = [_rows(EB, D)] + ([_rows(EB, DE)] if with_coord else [])
    out_shape = [jax.ShapeDtypeStruct((E_P, D), jnp.float32)] + (
        [jax.ShapeDtypeStruct((E_P, DE), jnp.float32)] if with_coord else [])
    f = pl.pallas_call(
        _make_edge_body(with_coord),
        grid=(E_P // EB,),
        in_specs=in_specs,
        out_specs=out_specs if n_out > 1 else out_specs[0],
        out_shape=out_shape if n_out > 1 else out_shape[0],
    )
    return f(*args)


def _make_node_body(n_partial, final):
    if not final:
        def body(h, hs, ms, xp, wnh, wnn, bn1, wn2, bn2, g, c, wa, wb,
                 h_ref, sa_ref, sb_ref):
            hsum = hs[0]
            msum = ms[0]
            for k in range(1, n_partial):
                hsum = hsum + hs[k]
                msum = msum + ms[k]
            cnt = jnp.maximum(msum[:, 3:4], 1.0)
            hn = _silu(_dot(h[...], wnh[...]) + _dot(hsum, wnn[...]) + bn1[...])
            hh = _dot(hn, wn2[...]) + bn2[...]
            h1 = _ln(hh, g[...], c[...])
            col = jax.lax.broadcasted_iota(jnp.int32, (1, DE), 1)
            m3 = jnp.where(col < 3, 1.0, 0.0)
            x2 = xp[...] + (msum * m3) / cnt
            h_ref[...] = h1
            sa_ref[...] = _pack(_dot(h1, wa[...]), x2)
            sb_ref[...] = _pack(_dot(h1, wb[...]), x2)
        return body
    else:
        def body(h, hs, wnh, wnn, bn1, wn2, bn2, wmu, bmu, wlv, blv, eps,
                 z_ref, mu_ref, lv_ref, zp_ref):
            hsum = hs[0]
            for k in range(1, n_partial):
                hsum = hsum + hs[k]
            hn = _silu(_dot(h[...], wnh[...]) + _dot(hsum, wnn[...]) + bn1[...])
            h2 = _dot(hn, wn2[...]) + bn2[...]
            mu = _dot(h2, wmu[...]) + bmu[...]
            lv = _dot(h2, wlv[...]) + blv[...]
            z = eps[...] * jnp.exp(lv) + mu
            z_ref[...] = z
            mu_ref[...] = mu
            lv_ref[...] = lv
            zp_ref[...] = jnp.concatenate([z, jnp.zeros_like(z)], axis=1)
        return body


def _part(nb, d, n_partial):
    return pl.BlockSpec((n_partial, nb, d), lambda i: (0, i, 0))


def _node_call(h, hs, ms, xp, wnh, wnn, bn1, wn2, bn2, g, c, wa, wb):
    np_ = hs.shape[0]
    f = pl.pallas_call(
        _make_node_body(np_, final=False),
        grid=(N_NODES // NB,),
        in_specs=[_rows(NB, D), _part(NB, D, np_), _part(NB, DE, np_),
                  _rows(NB, DE)] + [_full(a.shape) for a in
                                    (wnh, wnn, bn1, wn2, bn2, g, c, wa, wb)],
        out_specs=[_rows(NB, D), _rows(NB, W), _rows(NB, W)],
        out_shape=[jax.ShapeDtypeStruct((N_NODES, D), jnp.float32),
                   jax.ShapeDtypeStruct((N_NODES, W), jnp.int32),
                   jax.ShapeDtypeStruct((N_NODES, W), jnp.int32)],
    )
    return f(h, hs, ms, xp, wnh, wnn, bn1, wn2, bn2, g, c, wa, wb)


def _node_final_call(h, hs, wnh, wnn, bn1, wn2, bn2, wmu, bmu, wlv, blv, eps):
    np_ = hs.shape[0]
    f = pl.pallas_call(
        _make_node_body(np_, final=True),
        grid=(N_NODES // NB,),
        in_specs=[_rows(NB, D), _part(NB, D, np_)]
        + [_full(a.shape) for a in (wnh, wnn, bn1, wn2, bn2, wmu, bmu, wlv, blv)]
        + [_rows(NB, LAT)],
        out_specs=[_rows(NB, LAT)] * 3 + [_rows(NB, D)],
        out_shape=[jax.ShapeDtypeStruct((N_NODES, LAT), jnp.float32)] * 3
        + [jax.ShapeDtypeStruct((N_NODES, D), jnp.float32)],
    )
    return f(h, hs, wnh, wnn, bn1, wn2, bn2, wmu, bmu, wlv, blv, eps)


def _dec_body(zsp, zdp, zsn, zdn, w0, b0, w1, b1, pos_ref, neg_ref):
    xp = (zsp[:, :LAT] * zdp[:, :LAT]).astype(jnp.float32)
    xn = (zsn[:, :LAT] * zdn[:, :LAT]).astype(jnp.float32)
    ap = jax.nn.relu(_dot(xp, w0[...]) + b0[...])
    an = jax.nn.relu(_dot(xn, w0[...]) + b0[...])
    pos_ref[...] = jax.nn.sigmoid(_dot(ap, w1[...]) + b1[...])
    neg_ref[...] = jax.nn.sigmoid(_dot(an, w1[...]) + b1[...])


def _dec_call(zsp, zdp, zsn, zdn, w0, b0, w1, b1):
    f = pl.pallas_call(
        _dec_body,
        grid=(pl.cdiv(N_MASKED, MB),),
        in_specs=[_rows(MB, D)] * 4 + [_full(a.shape) for a in (w0, b0, w1, b1)],
        out_specs=[_rows(MB, D)] * 2,
        out_shape=[jax.ShapeDtypeStruct((N_MASKED, D), jnp.float32)] * 2,
    )
    return f(zsp, zdp, zsn, zdn, w0, b0, w1, b1)


def _sc_mesh():
    return plsc.VectorSubcoreMesh(core_axis_name="c", subcore_axis_name="s")


def _sc_gather_layer(sa, sb, src_g, dst_g):
    @functools.partial(
        pl.kernel, mesh=_sc_mesh(),
        out_type=[jax.ShapeDtypeStruct((E_P, W), jnp.int32),
                  jax.ShapeDtypeStruct((E_P, W), jnp.int32)],
        scratch_types=[
            pltpu.VMEM((E_SW, SG), jnp.int32),
            pltpu.VMEM((E_SW, SG), jnp.int32),
            pltpu.VMEM((SG, W), jnp.int32), pltpu.VMEM((SG, W), jnp.int32),
            pltpu.VMEM((SG, W), jnp.int32), pltpu.VMEM((SG, W), jnp.int32),
            pltpu.SemaphoreType.DMA, pltpu.SemaphoreType.DMA,
        ],
    )
    def k(sa_h, sb_h, sg_h, dg_h, oa, ob,
          si, di, a0, b0, a1, b1, gsem, ssem):
        wid = lax.axis_index("c") * NS + lax.axis_index("s")
        g0 = wid * E_SW
        pltpu.sync_copy(sg_h.at[wid], si)
        pltpu.sync_copy(dg_h.at[wid], di)

        def body(jj, _):
            j0 = 2 * jj
            j1 = j0 + 1
            cs = []
            for (j, ba, bb) in ((j0, a0, b0), (j1, a1, b1)):
                cs.append(pltpu.async_copy(sa_h.at[si.at[j]], ba, gsem))
                cs.append(pltpu.async_copy(sb_h.at[di.at[j]], bb, gsem))
            for c in cs:
                c.wait()
            ws = []
            for (j, ba, bb) in ((j0, a0, b0), (j1, a1, b1)):
                base = (g0 + j) * SG
                ws.append(pltpu.async_copy(ba, oa.at[pl.ds(base, SG)], ssem))
                ws.append(pltpu.async_copy(bb, ob.at[pl.ds(base, SG)], ssem))
            for w in ws:
                w.wait()
            return 0
        lax.fori_loop(0, E_SW // 2, body, 0)
    return k(sa, sb, src_g, dst_g)


def _sc_gather_dec(zp, g0_, g1_, g2_, g3_):
    @functools.partial(
        pl.kernel, mesh=_sc_mesh(),
        out_type=[jax.ShapeDtypeStruct((M_P, D), jnp.float32)] * 4,
        scratch_types=[pltpu.VMEM((M_GW, GL), jnp.int32)] * 4
        + [pltpu.VMEM((GL, D), jnp.float32)] * 4
        + [pltpu.SemaphoreType.DMA, pltpu.SemaphoreType.DMA],
    )
    def k(z_h, i0h, i1h, i2h, i3h, o0, o1, o2, o3,
          v0, v1, v2, v3, b0, b1, b2, b3, gsem, ssem):
        wid = lax.axis_index("c") * NS + lax.axis_index("s")
        g0 = wid * M_GW
        for ih, iv in ((i0h, v0), (i1h, v1), (i2h, v2), (i3h, v3)):
            pltpu.sync_copy(ih.at[wid], iv)

        def body(j, _):
            base = (g0 + j) * GL
            cs = [pltpu.async_copy(z_h.at[iv.at[j]], bb, gsem)
                  for iv, bb in ((v0, b0), (v1, b1), (v2, b2), (v3, b3))]
            for c in cs:
                c.wait()
            ws = [pltpu.async_copy(bb, oo.at[pl.ds(base, GL)], ssem)
                  for bb, oo in ((b0, o0), (b1, o1), (b2, o2), (b3, o3))]
            for w in ws:
                w.wait()
            return 0
        lax.fori_loop(0, M_GW, body, 0)
    return k(zp, g0_, g1_, g2_, g3_)


def _zero_vmem(ref, rows, cols):
    def zr(i, _):
        def zc(c, __):
            ref[i, pl.ds(c * 16, 16)] = jnp.zeros((16,), jnp.float32)
            return 0
        lax.fori_loop(0, cols // 16, zc, 0)
        return 0
    lax.fori_loop(0, rows, zr, 0)


def _make_scatter(with_mx):
    out_type = [jax.ShapeDtypeStruct((NC, N_PAD, D), jnp.float32)]
    scratch = [pltpu.VMEM((E_GW, GL), jnp.int32),
               pltpu.VMEM((GL, D), jnp.float32),
               pltpu.VMEM_SHARED((N_PAD, D), jnp.float32)]
    if with_mx:
        out_type.append(jax.ShapeDtypeStruct((NC, N_PAD, DE), jnp.float32))
        scratch += [pltpu.VMEM((GL, DE), jnp.float32),
                    pltpu.VMEM_SHARED((N_PAD, DE), jnp.float32)]

    def body_mx(m2_h, mx_h, dg_h, hpart, mpart,
                idx, buf, acc_h, bufx, acc_x):
        _scatter_common(m2_h, dg_h, hpart, idx, buf, acc_h,
                        mx_h, mpart, bufx, acc_x)

    def body_h(m2_h, dg_h, hpart, idx, buf, acc_h):
        _scatter_common(m2_h, dg_h, hpart, idx, buf, acc_h,
                        None, None, None, None)

    body = body_mx if with_mx else body_h
    return functools.partial(pl.kernel, mesh=_sc_mesh(), out_type=out_type,
                             scratch_types=scratch)(body)


def _scatter_common(m2_h, dg_h, hpart, idx, buf, acc_h,
                    mx_h, mpart, bufx, acc_x):
    cid = lax.axis_index("c")
    sid = lax.axis_index("s")
    wid = cid * NS + sid
    _zero_vmem(buf, RZ, D)
    if mx_h is not None:
        _zero_vmem(bufx, RZ, DE)
    for k5 in range(RT // RZ):
        rows = pl.ds(sid * RT + k5 * RZ, RZ)
        pltpu.sync_copy(buf, acc_h.at[rows])
        if mx_h is not None:
            pltpu.sync_copy(bufx, acc_x.at[rows])
    plsc.subcore_barrier()
    pltpu.sync_copy(dg_h.at[wid], idx)

    def body(j, _):
        base = (wid * E_GW + j) * GL
        pltpu.sync_copy(m2_h.at[pl.ds(base, GL)], buf)
        pltpu.sync_copy(buf, acc_h.at[idx.at[j]], add=True)
        if mx_h is not None:
            pltpu.sync_copy(mx_h.at[pl.ds(base, GL)], bufx)
            pltpu.sync_copy(bufx, acc_x.at[idx.at[j]], add=True)
        return 0
    lax.fori_loop(0, E_GW, body, 0)
    plsc.subcore_barrier()
    for k5 in range(RT // RZ):
        rows = pl.ds(sid * RT + k5 * RZ, RZ)
        pltpu.sync_copy(acc_h.at[rows], hpart.at[cid, rows])
        if mx_h is not None:
            pltpu.sync_copy(acc_x.at[rows], mpart.at[cid, rows])


@functools.lru_cache(maxsize=None)
def _scatter_kernel(with_mx):
    return _make_scatter(with_mx)


def _scatter_mx(m2, mx, dg):
    return _scatter_kernel(True)(m2, mx, dg)


def _scatter_h(m2, dg):
    return _scatter_kernel(False)(m2, dg)


def kernel(node_feat, xyz, edge_feat, edge_index, masked_edges, neg_edges,
           params):
    p = params
    t = lambda w: w.T
    row = lambda v: v.reshape(1, -1)

    src = edge_index[0].astype(jnp.int32)
    dst = edge_index[1].astype(jnp.int32)
    pad = E_P - N_EDGES
    srcp = jnp.concatenate([src, jnp.zeros((pad,), jnp.int32)])
    dstp = jnp.concatenate([dst, jnp.zeros((pad,), jnp.int32)])
    srcg = srcp.reshape(NW, E_SW, SG)
    dstg = dstp.reshape(NW, E_SW, SG)
    dstg_sc = dstp.reshape(NW, E_GW, GL)
    efp = jnp.pad(edge_feat, ((0, pad), (0, 0)))
    xp = jnp.pad(xyz, ((0, 0), (0, DE - 3)))

    mpad = M_P - N_MASKED
    def decg(e):
        return jnp.concatenate([e.astype(jnp.int32),
                                jnp.zeros((mpad,), jnp.int32)]
                               ).reshape(NW, M_GW, GL)

    e = p['emb1']; e2 = p['emb2']
    l1, l2 = p['egnn']

    def split_e1(lp):
        Wm = lp['e1']['W']
        return (row(Wm[:, 2 * D]), t(Wm[:, 2 * D + 1:]), row(lp['e1']['b']))

    h, sa, sb = _embed_call(
        node_feat, xp, t(e['W']), row(e['b']),
        row(p['emb_ln1']['g']), row(p['emb_ln1']['b']),
        t(e2['W']), row(e2['b']),
        row(p['emb_ln2']['g']), row(p['emb_ln2']['b']),
        t(l1['e1']['W'][:, :D]), t(l1['e1']['W'][:, D:2 * D]))

    for li, lp in enumerate((l1, l2)):
        wc_, wd_, be1_ = split_e1(lp)
        ga, gb = _sc_gather_layer(sa, sb, srcg, dstg)
        if li == 0:
            m2, mx = _edge_call(True, ga, gb, efp, wd_, wc_, be1_,
                                t(lp['e2']['W']), row(lp['e2']['b']),
                                t(lp['c1']['W']), row(lp['c1']['b']),
                                row(lp['c2']['W'][0]))
            s = jax.ops.segment_sum(
                jnp.concatenate([m2[:N_EDGES], mx[:N_EDGES]], axis=1),
                dst, num_segments=N_NODES)
            hsum = s[:, :D][None]
            msum = s[:, D:][None]
            n2p = p['egnn'][1]
            h, sa, sb = _node_call(
                h, hsum, msum, xp,
                t(lp['n1']['W'][:, :D]), t(lp['n1']['W'][:, D:]),
                row(lp['n1']['b']), t(lp['n2']['W']), row(lp['n2']['b']),
                row(p['gc_ln']['g']), row(p['gc_ln']['b']),
                t(n2p['e1']['W'][:, :D]), t(n2p['e1']['W'][:, D:2 * D]))
        else:
            m2 = _edge_call(False, ga, gb, efp, wd_, wc_, be1_,
                            t(lp['e2']['W']), row(lp['e2']['b']))
            hsum = jax.ops.segment_sum(m2[:N_EDGES], dst,
                                       num_segments=N_NODES)[None]
            eps = jax.random.uniform(jax.random.key(42), (N_NODES, LAT),
                                     dtype=jnp.float32)
            z, mu, lv, zp = _node_final_call(
                h, hsum,
                t(lp['n1']['W'][:, :D]), t(lp['n1']['W'][:, D:]),
                row(lp['n1']['b']), t(lp['n2']['W']), row(lp['n2']['b']),
                t(p['mu']['W']), row(p['mu']['b']),
                t(p['lv']['W']), row(p['lv']['b']), eps)

    zsp, zdp, zsn, zdn = _sc_gather_dec(
        zp, decg(masked_edges[0]), decg(masked_edges[1]),
        decg(neg_edges[0]), decg(neg_edges[1]))
    pos_out, neg_out = _dec_call(zsp, zdp, zsn, zdn,
                                 t(p['dec0']['W']), row(p['dec0']['b']),
                                 t(p['dec1']['W']), row(p['dec1']['b']))
    return (z, mu, lv, pos_out, neg_out)
